# Initial kernel scaffold; baseline (speedup 1.0000x reference)
import functools

import jax
import jax.numpy as jnp
from jax import lax
from jax.experimental import pallas as pl
from jax.experimental.pallas import tpu as pltpu
from jax.experimental.pallas import tpu_sc as plsc

B = 4096
NPG = 13
N = B * NPG
E = B * 24
NODE_DIM = 20
EDGE_DIM = 4
H = 64
HEADS = 4
NJ = 12
NR = 5

LN_EPS = 1e-5
SEG_EPS = 1e-16

_NBLK = 128
_EBLK = 512


def _stage_a1_body(x_ref, t_ref, wp_ref, bp_ref, wl_ref, bl_ref, wr_ref,
                   br_ref, xl_ref, xr_ref):
    xb = x_ref[...]
    tb = t_ref[...]
    acc = jnp.zeros((_NBLK, H), jnp.float32)
    for r in range(NR):
        pr = jnp.dot(xb, wp_ref[r], preferred_element_type=jnp.float32)
        pr = pr + bp_ref[r][None, :]
        acc = jnp.where(tb == r, pr, acc)
    h0 = jnp.where(acc > 0, acc, jnp.expm1(acc))
    yl = jnp.dot(h0, wl_ref[...], preferred_element_type=jnp.float32) + bl_ref[...]
    yr = jnp.dot(h0, wr_ref[...], preferred_element_type=jnp.float32) + br_ref[...]
    for h in range(HEADS):
        xl_ref[h] = yl[:, h * H:(h + 1) * H]
        xr_ref[h] = yr[:, h * H:(h + 1) * H]


def _stage_a1(x, types2d, Wp, bp, Wl1, bl1, Wr1, br1):
    grid = (N // _NBLK,)
    return pl.pallas_call(
        _stage_a1_body,
        grid=grid,
        in_specs=[
            pl.BlockSpec((_NBLK, NODE_DIM), lambda i: (i, 0)),
            pl.BlockSpec((_NBLK, 1), lambda i: (i, 0)),
            pl.BlockSpec((NR, NODE_DIM, H), lambda i: (0, 0, 0)),
            pl.BlockSpec((NR, H), lambda i: (0, 0)),
            pl.BlockSpec((H, H * HEADS), lambda i: (0, 0)),
            pl.BlockSpec((1, H * HEADS), lambda i: (0, 0)),
            pl.BlockSpec((H, H * HEADS), lambda i: (0, 0)),
            pl.BlockSpec((1, H * HEADS), lambda i: (0, 0)),
        ],
        out_specs=[
            pl.BlockSpec((HEADS, _NBLK, H), lambda i: (0, i, 0)),
            pl.BlockSpec((HEADS, _NBLK, H), lambda i: (0, i, 0)),
        ],
        out_shape=[
            jax.ShapeDtypeStruct((HEADS, N, H), jnp.float32),
            jax.ShapeDtypeStruct((HEADS, N, H), jnp.float32),
        ],
    )(x, types2d, Wp, bp, Wl1, bl1, Wr1, br1)


def _stage_a2_body(ea_ref, we1_ref, we2_ref, e1_ref, e2_ref):
    eb = ea_ref[...]
    y1 = jnp.dot(eb, we1_ref[...], preferred_element_type=jnp.float32)
    for h in range(HEADS):
        e1_ref[h] = y1[:, h * H:(h + 1) * H]
    e2_ref[...] = jnp.dot(eb, we2_ref[...], preferred_element_type=jnp.float32)


def _stage_a2(ea, We1, We2):
    grid = (E // _EBLK,)
    return pl.pallas_call(
        _stage_a2_body,
        grid=grid,
        in_specs=[
            pl.BlockSpec((_EBLK, EDGE_DIM), lambda i: (i, 0)),
            pl.BlockSpec((EDGE_DIM, H * HEADS), lambda i: (0, 0)),
            pl.BlockSpec((EDGE_DIM, H), lambda i: (0, 0)),
        ],
        out_specs=[
            pl.BlockSpec((HEADS, _EBLK, H), lambda i: (0, i, 0)),
            pl.BlockSpec((_EBLK, H), lambda i: (i, 0)),
        ],
        out_shape=[
            jax.ShapeDtypeStruct((HEADS, E, H), jnp.float32),
            jax.ShapeDtypeStruct((E, H), jnp.float32),
        ],
    )(ea, We1, We2)


_K = 128
_NHALF = N // 2
_RPT = _NHALF // 16
_DENW = 8


def _sc_edge_body(nheads, nedges, xl_hbm, xr_hbm, e_hbm, src_hbm, dst_hbm,
                  att_hbm, z64_hbm, z8_hbm, acc_out, den_out,
                  acc_s, den_s, attb, srcb, dstb, gsrc, gdst, ldst,
                  xlb, xrb, ebuf, stage, dstage, sem1, sem2):
    cid = lax.axis_index("c")
    sid = lax.axis_index("s")
    ept = nedges // 16
    nch = ept // _K
    tile_base = sid * ept
    iota16 = lax.iota(jnp.int32, 16)
    den_mask = iota16 < _DENW

    pltpu.sync_copy(att_hbm, attb)

    for h in range(nheads):
        pltpu.sync_copy(z64_hbm, acc_s.at[pl.ds(sid * _RPT, _RPT)])
        pltpu.sync_copy(z8_hbm, den_s.at[pl.ds(sid * _RPT, _RPT)])

        @pl.when(sid == 0)
        def _():
            pltpu.sync_copy(z64_hbm.at[pl.ds(0, 8)],
                            acc_s.at[pl.ds(_NHALF, 8)])
            pltpu.sync_copy(z8_hbm.at[pl.ds(0, 8)],
                            den_s.at[pl.ds(_NHALF, 8)])

        plsc.subcore_barrier()

        att_v = [attb[h, pl.ds(j * 16, 16)] for j in range(H // 16)]
        row_off = jnp.int32(h * (xl_hbm.shape[0] // nheads))

        def chunk_body(c, carry):
            estart = pl.multiple_of(tile_base + c * _K, _K)
            pltpu.sync_copy(src_hbm.at[pl.ds(estart, _K)], srcb)
            pltpu.sync_copy(dst_hbm.at[pl.ds(estart, _K)], dstb)
            for j in range(_K // 16):
                sl = pl.ds(j * 16, 16)
                sv = srcb[sl]
                dv = dstb[sl]
                gsrc[sl] = sv + row_off
                gdst[sl] = dv + row_off
                dl = dv - cid * _NHALF
                ok = (dl >= 0) & (dl < _NHALF)
                ldst[sl] = jnp.where(ok, dl, _NHALF)
            cp1 = pltpu.async_copy(xl_hbm.at[gsrc], xlb, sem1)
            cp2 = pltpu.async_copy(xr_hbm.at[gdst], xrb, sem2)
            pltpu.sync_copy(
                e_hbm.at[pl.ds(pl.multiple_of(estart + h * nedges, _K), _K)],
                ebuf)
            cp1.wait()
            cp2.wait()

            def edge_body(e, carry2):
                tsum = jnp.zeros((16,), jnp.float32)
                xlv = []
                for j in range(H // 16):
                    sl = pl.ds(j * 16, 16)
                    xv = xlb[e, sl]
                    xlv.append(xv)
                    u = xv + xrb[e, sl] + ebuf[e, sl]
                    m = jnp.where(u >= 0, u, 0.2 * u)
                    tsum = tsum + m * att_v[j]
                al = jnp.sum(tsum)
                exv = jnp.exp(jnp.broadcast_to(al, (16,)))
                for j in range(H // 16):
                    stage[e, pl.ds(j * 16, 16)] = xlv[j] * exv
                rowi = jnp.full((16,), e, jnp.int32)
                plsc.store_scatter(dstage, [rowi, iota16], exv, mask=den_mask)
                return carry2

            lax.fori_loop(0, _K, edge_body, 0, unroll=2)
            pltpu.sync_copy(stage, acc_s.at[ldst], add=True)
            pltpu.sync_copy(dstage, den_s.at[ldst], add=True)
            return carry

        lax.fori_loop(0, nch, chunk_body, 0)

        plsc.subcore_barrier()

        out_base = h * (xl_hbm.shape[0] // nheads) + cid * _NHALF + sid * _RPT
        pltpu.sync_copy(acc_s.at[pl.ds(sid * _RPT, _RPT)],
                        acc_out.at[pl.ds(pl.multiple_of(out_base, 8), _RPT)])
        pltpu.sync_copy(den_s.at[pl.ds(sid * _RPT, _RPT)],
                        den_out.at[pl.ds(pl.multiple_of(out_base, 8), _RPT)])
        plsc.subcore_barrier()


def _sc_edge(nheads, xl2d, xr2d, e2d, src, dst, att):
    mesh = plsc.VectorSubcoreMesh(core_axis_name="c", subcore_axis_name="s")
    z64 = jnp.zeros((_RPT, H), jnp.float32)
    z8 = jnp.zeros((_RPT, _DENW), jnp.float32)
    fn = pl.kernel(
        functools.partial(_sc_edge_body, nheads, E),
        out_type=[
            jax.ShapeDtypeStruct((nheads * N, H), jnp.float32),
            jax.ShapeDtypeStruct((nheads * N, _DENW), jnp.float32),
        ],
        mesh=mesh,
        scratch_types=[
            pltpu.VMEM_SHARED((_NHALF + 8, H), jnp.float32),
            pltpu.VMEM_SHARED((_NHALF + 8, _DENW), jnp.float32),
            pltpu.VMEM((nheads, H), jnp.float32),
            pltpu.VMEM((_K,), jnp.int32),
            pltpu.VMEM((_K,), jnp.int32),
            pltpu.VMEM((_K,), jnp.int32),
            pltpu.VMEM((_K,), jnp.int32),
            pltpu.VMEM((_K,), jnp.int32),
            pltpu.VMEM((_K, H), jnp.float32),
            pltpu.VMEM((_K, H), jnp.float32),
            pltpu.VMEM((_K, H), jnp.float32),
            pltpu.VMEM((_K, H), jnp.float32),
            pltpu.VMEM((_K, _DENW), jnp.float32),
            pltpu.SemaphoreType.DMA,
            pltpu.SemaphoreType.DMA,
        ],
    )
    return fn(xl2d, xr2d, e2d, src, dst, att, z64, z8)


def _stage_c_body(acc_ref, den_ref, b1_ref, g1_ref, be1_ref, wl2_ref, bl2_ref,
                  wr2_ref, br2_ref, xl2_ref, xr2_ref):
    parts = []
    for h in range(HEADS):
        d = den_ref[h][:, 0:1]
        parts.append(acc_ref[h] / (d + SEG_EPS))
    o = jnp.concatenate(parts, axis=1) + b1_ref[...]
    mu = o.mean(-1, keepdims=True)
    v = ((o - mu) ** 2).mean(-1, keepdims=True)
    y = (o - mu) / jnp.sqrt(v + LN_EPS) * g1_ref[...] + be1_ref[...]
    h1 = jnp.where(y > 0, y, jnp.expm1(y))
    xl2_ref[...] = jnp.dot(h1, wl2_ref[...],
                           preferred_element_type=jnp.float32) + bl2_ref[...]
    xr2_ref[...] = jnp.dot(h1, wr2_ref[...],
                           preferred_element_type=jnp.float32) + br2_ref[...]


def _stage_c(acc1, den1, bias1, g1, be1, Wl2, bl2, Wr2, br2):
    grid = (N // _NBLK,)
    return pl.pallas_call(
        _stage_c_body,
        grid=grid,
        in_specs=[
            pl.BlockSpec((HEADS, _NBLK, H), lambda i: (0, i, 0)),
            pl.BlockSpec((HEADS, _NBLK, _DENW), lambda i: (0, i, 0)),
            pl.BlockSpec((1, H * HEADS), lambda i: (0, 0)),
            pl.BlockSpec((1, H * HEADS), lambda i: (0, 0)),
            pl.BlockSpec((1, H * HEADS), lambda i: (0, 0)),
            pl.BlockSpec((H * HEADS, H), lambda i: (0, 0)),
            pl.BlockSpec((1, H), lambda i: (0, 0)),
            pl.BlockSpec((H * HEADS, H), lambda i: (0, 0)),
            pl.BlockSpec((1, H), lambda i: (0, 0)),
        ],
        out_specs=[
            pl.BlockSpec((_NBLK, H), lambda i: (i, 0)),
            pl.BlockSpec((_NBLK, H), lambda i: (i, 0)),
        ],
        out_shape=[
            jax.ShapeDtypeStruct((N, H), jnp.float32),
            jax.ShapeDtypeStruct((N, H), jnp.float32),
        ],
    )(acc1, den1, bias1, g1, be1, Wl2, bl2, Wr2, br2)


_GBLK = 128
_NROWS = _GBLK * NPG
_LOG2PI = 1.8378770664093453


def _stage_e_body(acc_ref, den_ref, b2_ref, g2_ref, be2_ref, act_ref, ls_ref,
                  a1w_ref, a1b_ref, a2w_ref, a2b_ref, c1w_ref, c1b_ref,
                  c2w_ref, c2b_ref, lp_ref, ent_ref, val_ref):
    d = den_ref[...][:, 0:1]
    o = acc_ref[...] / (d + SEG_EPS) + b2_ref[...]
    mu = o.mean(-1, keepdims=True)
    v = ((o - mu) ** 2).mean(-1, keepdims=True)
    y = (o - mu) / jnp.sqrt(v + LN_EPS) * g2_ref[...] + be2_ref[...]
    hf = jnp.where(y > 0, y, jnp.expm1(y))
    hr = hf.reshape(_GBLK, NPG, H)
    pooled = hr.mean(axis=1)
    jh = hr[:, 1:, :].reshape(_GBLK * NJ, H)
    t = jnp.tanh(jnp.dot(jh, a1w_ref[...],
                         preferred_element_type=jnp.float32) + a1b_ref[...])
    mean = (t * a2w_ref[...]).sum(-1).reshape(_GBLK, NJ) + a2b_ref[0, 0]
    std = jnp.clip(jnp.exp(ls_ref[...]), 0.15, 0.8)
    av = act_ref[...]
    lp = (-((av - mean) ** 2) / (2.0 * std ** 2) - jnp.log(std)
          - 0.5 * _LOG2PI)
    lp_ref[...] = lp.sum(-1, keepdims=True)
    ent = (0.5 + 0.5 * _LOG2PI + jnp.log(std)).sum()
    ent_ref[...] = jnp.broadcast_to(ent, (_GBLK, 1))
    tv = jnp.tanh(jnp.dot(pooled, c1w_ref[...],
                          preferred_element_type=jnp.float32) + c1b_ref[...])
    val_ref[...] = (tv * c2w_ref[...]).sum(-1, keepdims=True) + c2b_ref[0, 0]


def _stage_e(acc2, den2, bias2, g2, be2, action, log_std, A1, a1, A2t, a2,
             C1, c1, C2t, c2):
    grid = (B // _GBLK,)
    return pl.pallas_call(
        _stage_e_body,
        grid=grid,
        in_specs=[
            pl.BlockSpec((_NROWS, H), lambda i: (i, 0)),
            pl.BlockSpec((_NROWS, _DENW), lambda i: (i, 0)),
            pl.BlockSpec((1, H), lambda i: (0, 0)),
            pl.BlockSpec((1, H), lambda i: (0, 0)),
            pl.BlockSpec((1, H), lambda i: (0, 0)),
            pl.BlockSpec((_GBLK, NJ), lambda i: (i, 0)),
            pl.BlockSpec((1, NJ), lambda i: (0, 0)),
            pl.BlockSpec((H, 64), lambda i: (0, 0)),
            pl.BlockSpec((1, 64), lambda i: (0, 0)),
            pl.BlockSpec((1, 64), lambda i: (0, 0)),
            pl.BlockSpec((1, 1), lambda i: (0, 0)),
            pl.BlockSpec((H, 64), lambda i: (0, 0)),
            pl.BlockSpec((1, 64), lambda i: (0, 0)),
            pl.BlockSpec((1, 64), lambda i: (0, 0)),
            pl.BlockSpec((1, 1), lambda i: (0, 0)),
        ],
        out_specs=[
            pl.BlockSpec((_GBLK, 1), lambda i: (i, 0)),
            pl.BlockSpec((_GBLK, 1), lambda i: (i, 0)),
            pl.BlockSpec((_GBLK, 1), lambda i: (i, 0)),
        ],
        out_shape=[
            jax.ShapeDtypeStruct((B, 1), jnp.float32),
            jax.ShapeDtypeStruct((B, 1), jnp.float32),
            jax.ShapeDtypeStruct((B, 1), jnp.float32),
        ],
    )(acc2, den2, bias2, g2, be2, action, log_std, A1, a1, A2t, a2,
      C1, c1, C2t, c2)


def kernel(x, edge_index, edge_attr, node_types, batch, action, Wp, bp, Wl1,
           bl1, Wr1, br1, We1, att1, bias1, g1, be1, Wl2, bl2, Wr2, br2, We2,
           att2, bias2, g2, be2, A1, a1, A2, a2, C1, c1, C2, c2, log_std):
    src = edge_index[0]
    dst = edge_index[1]
    types2d = node_types.reshape(N, 1).astype(jnp.int32)

    xl1, xr1 = _stage_a1(x, types2d, Wp, bp, Wl1, bl1.reshape(1, -1), Wr1,
                         br1.reshape(1, -1))
    e1, e2 = _stage_a2(edge_attr, We1, We2)

    acc1, den1 = _sc_edge(HEADS, xl1.reshape(HEADS * N, H),
                          xr1.reshape(HEADS * N, H),
                          e1.reshape(HEADS * E, H), src, dst, att1)

    xl2, xr2 = _stage_c(acc1.reshape(HEADS, N, H),
                        den1.reshape(HEADS, N, _DENW), bias1.reshape(1, -1),
                        g1.reshape(1, -1), be1.reshape(1, -1), Wl2,
                        bl2.reshape(1, -1), Wr2, br2.reshape(1, -1))

    acc2, den2 = _sc_edge(1, xl2, xr2, e2, src, dst, att2)

    lp, ent, val = _stage_e(acc2, den2, bias2.reshape(1, -1),
                            g2.reshape(1, -1), be2.reshape(1, -1), action,
                            log_std.reshape(1, -1), A1, a1.reshape(1, -1),
                            A2.reshape(1, -1), a2.reshape(1, 1), C1,
                            c1.reshape(1, -1), C2.reshape(1, -1),
                            c2.reshape(1, 1))

    return (action, lp.reshape(B), ent.reshape(B), val)


# speedup vs baseline: 3.7112x; 3.7112x over previous
import functools

import jax
import jax.numpy as jnp
from jax import lax
from jax.experimental import pallas as pl
from jax.experimental.pallas import tpu as pltpu
from jax.experimental.pallas import tpu_sc as plsc

B = 4096
NPG = 13
N = B * NPG
E = B * 24
NODE_DIM = 20
EDGE_DIM = 4
H = 64
HEADS = 4
NJ = 12
NR = 5

LN_EPS = 1e-5
SEG_EPS = 1e-16

_NBLK = 128
_EBLK = 512


def _stage_a1_body(x_ref, t_ref, wp_ref, bp_ref, wl_ref, bl_ref, wr_ref,
                   br_ref, xl_ref, xr_ref):
    xb = x_ref[...]
    tb = t_ref[...]
    acc = jnp.zeros((_NBLK, H), jnp.float32)
    for r in range(NR):
        pr = jnp.dot(xb, wp_ref[r], preferred_element_type=jnp.float32)
        pr = pr + bp_ref[r][None, :]
        acc = jnp.where(tb == r, pr, acc)
    h0 = jnp.where(acc > 0, acc, jnp.exp(acc) - 1.0)
    yl = jnp.dot(h0, wl_ref[...], preferred_element_type=jnp.float32) + bl_ref[...]
    yr = jnp.dot(h0, wr_ref[...], preferred_element_type=jnp.float32) + br_ref[...]
    for h in range(HEADS):
        xl_ref[h] = yl[:, h * H:(h + 1) * H]
        xr_ref[h] = yr[:, h * H:(h + 1) * H]


def _stage_a1(x, types2d, Wp, bp, Wl1, bl1, Wr1, br1):
    grid = (N // _NBLK,)
    return pl.pallas_call(
        _stage_a1_body,
        grid=grid,
        in_specs=[
            pl.BlockSpec((_NBLK, NODE_DIM), lambda i: (i, 0)),
            pl.BlockSpec((_NBLK, 1), lambda i: (i, 0)),
            pl.BlockSpec((NR, NODE_DIM, H), lambda i: (0, 0, 0)),
            pl.BlockSpec((NR, H), lambda i: (0, 0)),
            pl.BlockSpec((H, H * HEADS), lambda i: (0, 0)),
            pl.BlockSpec((1, H * HEADS), lambda i: (0, 0)),
            pl.BlockSpec((H, H * HEADS), lambda i: (0, 0)),
            pl.BlockSpec((1, H * HEADS), lambda i: (0, 0)),
        ],
        out_specs=[
            pl.BlockSpec((HEADS, _NBLK, H), lambda i: (0, i, 0)),
            pl.BlockSpec((HEADS, _NBLK, H), lambda i: (0, i, 0)),
        ],
        out_shape=[
            jax.ShapeDtypeStruct((HEADS, N, H), jnp.float32),
            jax.ShapeDtypeStruct((HEADS, N, H), jnp.float32),
        ],
    )(x, types2d, Wp, bp, Wl1, bl1, Wr1, br1)


def _stage_a2_body(ea_ref, we1_ref, we2_ref, e1_ref, e2_ref):
    eb = ea_ref[...]
    y1 = jnp.dot(eb, we1_ref[...], preferred_element_type=jnp.float32)
    for h in range(HEADS):
        e1_ref[h] = y1[:, h * H:(h + 1) * H]
    e2_ref[...] = jnp.dot(eb, we2_ref[...], preferred_element_type=jnp.float32)


def _stage_a2(ea, We1, We2):
    grid = (E // _EBLK,)
    return pl.pallas_call(
        _stage_a2_body,
        grid=grid,
        in_specs=[
            pl.BlockSpec((_EBLK, EDGE_DIM), lambda i: (i, 0)),
            pl.BlockSpec((EDGE_DIM, H * HEADS), lambda i: (0, 0)),
            pl.BlockSpec((EDGE_DIM, H), lambda i: (0, 0)),
        ],
        out_specs=[
            pl.BlockSpec((HEADS, _EBLK, H), lambda i: (0, i, 0)),
            pl.BlockSpec((_EBLK, H), lambda i: (i, 0)),
        ],
        out_shape=[
            jax.ShapeDtypeStruct((HEADS, E, H), jnp.float32),
            jax.ShapeDtypeStruct((E, H), jnp.float32),
        ],
    )(ea, We1, We2)


_K = 64
_NHALF = N // 2
_RPT = _NHALF // 16
_DENW = 4


def _sc_edge_body(nheads, nedges, xl_hbm, xr_hbm, e_hbm, src_hbm, dst_hbm,
                  att_hbm, z64_hbm, z8_hbm, acc_out, den_out,
                  acc_s, den_s, attb, srcb, dstb, gsrc, gdst, ldst,
                  xlb, xrb, ebuf, stage, dstage, scr, sem1, sem2):
    cid = lax.axis_index("c")
    sid = lax.axis_index("s")
    ept = nedges // 16
    nch = ept // _K
    tile_base = sid * ept
    iota16 = lax.iota(jnp.int32, 16)
    den_mask = iota16 < _DENW

    pltpu.sync_copy(att_hbm, attb)

    for h in range(nheads):
        pltpu.sync_copy(z64_hbm, acc_s.at[pl.ds(sid * _RPT, _RPT)])
        pltpu.sync_copy(z8_hbm, den_s.at[pl.ds(sid * _RPT, _RPT)])

        @pl.when(sid == 0)
        def _():
            pltpu.sync_copy(z64_hbm.at[pl.ds(0, 8)],
                            acc_s.at[pl.ds(_NHALF, 8)])
            pltpu.sync_copy(z8_hbm.at[pl.ds(0, 8)],
                            den_s.at[pl.ds(_NHALF, 8)])

        plsc.subcore_barrier()

        att_v = [attb[h, pl.ds(j * 16, 16)] for j in range(H // 16)]
        row_off = jnp.int32(h * (xl_hbm.shape[0] // nheads))

        def chunk_body(c, carry):
            estart = pl.multiple_of(tile_base + c * _K, _K)
            pltpu.sync_copy(src_hbm.at[pl.ds(estart, _K)], srcb)
            pltpu.sync_copy(dst_hbm.at[pl.ds(estart, _K)], dstb)
            for j in range(_K // 16):
                sl = pl.ds(j * 16, 16)
                sv = srcb[sl]
                dv = dstb[sl]
                gsrc[sl] = sv + row_off
                gdst[sl] = dv + row_off
                dl = dv - cid * _NHALF
                ok = (dl >= 0) & (dl < _NHALF)
                ldst[sl] = jnp.where(ok, dl, _NHALF)
            cp1 = pltpu.async_copy(xl_hbm.at[gsrc], xlb, sem1)
            cp2 = pltpu.async_copy(xr_hbm.at[gdst], xrb, sem2)
            pltpu.sync_copy(
                e_hbm.at[pl.ds(pl.multiple_of(estart + h * nedges, _K), _K)],
                ebuf)
            cp1.wait()
            cp2.wait()

            def edge_body(e, carry2):
                tsum = jnp.zeros((16,), jnp.float32)
                xlv = []
                for j in range(H // 16):
                    sl = pl.ds(j * 16, 16)
                    xv = xlb[e, sl]
                    xlv.append(xv)
                    u = xv + xrb[e, sl] + ebuf[e, sl]
                    m = jnp.where(u >= 0, u, 0.2 * u)
                    tsum = tsum + m * att_v[j]
                for sh in (1, 2, 4, 8):
                    scr[...] = tsum
                    perm = jnp.bitwise_xor(iota16, sh)
                    tsum = tsum + plsc.load_gather(scr, [perm])
                exv = jnp.exp(tsum)
                for j in range(H // 16):
                    stage[e, pl.ds(j * 16, 16)] = xlv[j] * exv
                rowi = jnp.full((16,), e, jnp.int32)
                plsc.store_scatter(dstage, [rowi, iota16], exv, mask=den_mask)
                return carry2

            lax.fori_loop(0, _K, edge_body, 0, unroll=2)
            pltpu.sync_copy(stage, acc_s.at[ldst], add=True)
            pltpu.sync_copy(dstage, den_s.at[ldst], add=True)
            return carry

        lax.fori_loop(0, nch, chunk_body, 0)

        plsc.subcore_barrier()

        out_base = h * (xl_hbm.shape[0] // nheads) + cid * _NHALF + sid * _RPT
        pltpu.sync_copy(acc_s.at[pl.ds(sid * _RPT, _RPT)],
                        acc_out.at[pl.ds(pl.multiple_of(out_base, 8), _RPT)])
        pltpu.sync_copy(den_s.at[pl.ds(sid * _RPT, _RPT)],
                        den_out.at[pl.ds(pl.multiple_of(out_base, 8), _RPT)])
        plsc.subcore_barrier()


def _sc_edge(nheads, xl2d, xr2d, e2d, src, dst, att):
    mesh = plsc.VectorSubcoreMesh(core_axis_name="c", subcore_axis_name="s")
    z64 = jnp.zeros((_RPT, H), jnp.float32)
    z8 = jnp.zeros((_RPT, _DENW), jnp.float32)
    fn = pl.kernel(
        functools.partial(_sc_edge_body, nheads, E),
        out_type=[
            jax.ShapeDtypeStruct((nheads * N, H), jnp.float32),
            jax.ShapeDtypeStruct((nheads * N, _DENW), jnp.float32),
        ],
        mesh=mesh,
        compiler_params=pltpu.CompilerParams(
            needs_layout_passes=False, use_tc_tiling_on_sc=False),
        scratch_types=[
            pltpu.VMEM_SHARED((_NHALF + 8, H), jnp.float32),
            pltpu.VMEM_SHARED((_NHALF + 8, _DENW), jnp.float32),
            pltpu.VMEM((nheads, H), jnp.float32),
            pltpu.VMEM((_K,), jnp.int32),
            pltpu.VMEM((_K,), jnp.int32),
            pltpu.VMEM((_K,), jnp.int32),
            pltpu.VMEM((_K,), jnp.int32),
            pltpu.VMEM((_K,), jnp.int32),
            pltpu.VMEM((_K, H), jnp.float32),
            pltpu.VMEM((_K, H), jnp.float32),
            pltpu.VMEM((_K, H), jnp.float32),
            pltpu.VMEM((_K, H), jnp.float32),
            pltpu.VMEM((_K, _DENW), jnp.float32),
            pltpu.VMEM((16,), jnp.float32),
            pltpu.SemaphoreType.DMA,
            pltpu.SemaphoreType.DMA,
        ],
    )
    return fn(xl2d, xr2d, e2d, src, dst, att, z64, z8)


def _stage_c_body(acc_ref, den_ref, b1_ref, g1_ref, be1_ref, wl2_ref, bl2_ref,
                  wr2_ref, br2_ref, xl2_ref, xr2_ref):
    parts = []
    for h in range(HEADS):
        d = den_ref[h][:, 0:1]
        parts.append(acc_ref[h] / (d + SEG_EPS))
    o = jnp.concatenate(parts, axis=1) + b1_ref[...]
    mu = o.mean(-1, keepdims=True)
    v = ((o - mu) ** 2).mean(-1, keepdims=True)
    y = (o - mu) / jnp.sqrt(v + LN_EPS) * g1_ref[...] + be1_ref[...]
    h1 = jnp.where(y > 0, y, jnp.exp(y) - 1.0)
    xl2_ref[...] = jnp.dot(h1, wl2_ref[...],
                           preferred_element_type=jnp.float32) + bl2_ref[...]
    xr2_ref[...] = jnp.dot(h1, wr2_ref[...],
                           preferred_element_type=jnp.float32) + br2_ref[...]


def _stage_c(acc1, den1, bias1, g1, be1, Wl2, bl2, Wr2, br2):
    grid = (N // _NBLK,)
    return pl.pallas_call(
        _stage_c_body,
        grid=grid,
        in_specs=[
            pl.BlockSpec((HEADS, _NBLK, H), lambda i: (0, i, 0)),
            pl.BlockSpec((HEADS, _NBLK, _DENW), lambda i: (0, i, 0)),
            pl.BlockSpec((1, H * HEADS), lambda i: (0, 0)),
            pl.BlockSpec((1, H * HEADS), lambda i: (0, 0)),
            pl.BlockSpec((1, H * HEADS), lambda i: (0, 0)),
            pl.BlockSpec((H * HEADS, H), lambda i: (0, 0)),
            pl.BlockSpec((1, H), lambda i: (0, 0)),
            pl.BlockSpec((H * HEADS, H), lambda i: (0, 0)),
            pl.BlockSpec((1, H), lambda i: (0, 0)),
        ],
        out_specs=[
            pl.BlockSpec((_NBLK, H), lambda i: (i, 0)),
            pl.BlockSpec((_NBLK, H), lambda i: (i, 0)),
        ],
        out_shape=[
            jax.ShapeDtypeStruct((N, H), jnp.float32),
            jax.ShapeDtypeStruct((N, H), jnp.float32),
        ],
    )(acc1, den1, bias1, g1, be1, Wl2, bl2, Wr2, br2)


_GBLK = 128
_NROWS = _GBLK * NPG
_LOG2PI = 1.8378770664093453


def _stage_e_body(acc_ref, den_ref, b2_ref, g2_ref, be2_ref, act_ref, ls_ref,
                  a1w_ref, a1b_ref, a2w_ref, a2b_ref, c1w_ref, c1b_ref,
                  c2w_ref, c2b_ref, lp_ref, ent_ref, val_ref):
    d = den_ref[...][:, 0:1]
    o = acc_ref[...] / (d + SEG_EPS) + b2_ref[...]
    mu = o.mean(-1, keepdims=True)
    v = ((o - mu) ** 2).mean(-1, keepdims=True)
    y = (o - mu) / jnp.sqrt(v + LN_EPS) * g2_ref[...] + be2_ref[...]
    hf = jnp.where(y > 0, y, jnp.exp(y) - 1.0)
    hr = hf.reshape(_GBLK, NPG, H)
    pooled = hr.mean(axis=1)
    jh = hr[:, 1:, :].reshape(_GBLK * NJ, H)
    t = jnp.tanh(jnp.dot(jh, a1w_ref[...],
                         preferred_element_type=jnp.float32) + a1b_ref[...])
    mean = (t * a2w_ref[...]).sum(-1).reshape(_GBLK, NJ) + a2b_ref[0, 0]
    std = jnp.clip(jnp.exp(ls_ref[...]), 0.15, 0.8)
    av = act_ref[...]
    lp = (-((av - mean) ** 2) / (2.0 * std ** 2) - jnp.log(std)
          - 0.5 * _LOG2PI)
    lp_ref[...] = lp.sum(-1, keepdims=True)
    ent = (0.5 + 0.5 * _LOG2PI + jnp.log(std)).sum()
    ent_ref[...] = jnp.broadcast_to(ent, (_GBLK, 1))
    tv = jnp.tanh(jnp.dot(pooled, c1w_ref[...],
                          preferred_element_type=jnp.float32) + c1b_ref[...])
    val_ref[...] = (tv * c2w_ref[...]).sum(-1, keepdims=True) + c2b_ref[0, 0]


def _stage_e(acc2, den2, bias2, g2, be2, action, log_std, A1, a1, A2t, a2,
             C1, c1, C2t, c2):
    grid = (B // _GBLK,)
    return pl.pallas_call(
        _stage_e_body,
        grid=grid,
        in_specs=[
            pl.BlockSpec((_NROWS, H), lambda i: (i, 0)),
            pl.BlockSpec((_NROWS, _DENW), lambda i: (i, 0)),
            pl.BlockSpec((1, H), lambda i: (0, 0)),
            pl.BlockSpec((1, H), lambda i: (0, 0)),
            pl.BlockSpec((1, H), lambda i: (0, 0)),
            pl.BlockSpec((_GBLK, NJ), lambda i: (i, 0)),
            pl.BlockSpec((1, NJ), lambda i: (0, 0)),
            pl.BlockSpec((H, 64), lambda i: (0, 0)),
            pl.BlockSpec((1, 64), lambda i: (0, 0)),
            pl.BlockSpec((1, 64), lambda i: (0, 0)),
            pl.BlockSpec((1, 1), lambda i: (0, 0)),
            pl.BlockSpec((H, 64), lambda i: (0, 0)),
            pl.BlockSpec((1, 64), lambda i: (0, 0)),
            pl.BlockSpec((1, 64), lambda i: (0, 0)),
            pl.BlockSpec((1, 1), lambda i: (0, 0)),
        ],
        out_specs=[
            pl.BlockSpec((_GBLK, 1), lambda i: (i, 0)),
            pl.BlockSpec((_GBLK, 1), lambda i: (i, 0)),
            pl.BlockSpec((_GBLK, 1), lambda i: (i, 0)),
        ],
        out_shape=[
            jax.ShapeDtypeStruct((B, 1), jnp.float32),
            jax.ShapeDtypeStruct((B, 1), jnp.float32),
            jax.ShapeDtypeStruct((B, 1), jnp.float32),
        ],
    )(acc2, den2, bias2, g2, be2, action, log_std, A1, a1, A2t, a2,
      C1, c1, C2t, c2)


def kernel(x, edge_index, edge_attr, node_types, batch, action, Wp, bp, Wl1,
           bl1, Wr1, br1, We1, att1, bias1, g1, be1, Wl2, bl2, Wr2, br2, We2,
           att2, bias2, g2, be2, A1, a1, A2, a2, C1, c1, C2, c2, log_std):
    src = edge_index[0]
    dst = edge_index[1]
    types2d = node_types.reshape(N, 1).astype(jnp.int32)

    xl1, xr1 = _stage_a1(x, types2d, Wp, bp, Wl1, bl1.reshape(1, -1), Wr1,
                         br1.reshape(1, -1))
    e1, e2 = _stage_a2(edge_attr, We1, We2)

    acc1, den1 = _sc_edge(HEADS, xl1.reshape(HEADS * N, H),
                          xr1.reshape(HEADS * N, H),
                          e1.reshape(HEADS * E, H), src, dst, att1)

    xl2, xr2 = _stage_c(acc1.reshape(HEADS, N, H),
                        den1.reshape(HEADS, N, _DENW), bias1.reshape(1, -1),
                        g1.reshape(1, -1), be1.reshape(1, -1), Wl2,
                        bl2.reshape(1, -1), Wr2, br2.reshape(1, -1))

    acc2, den2 = _sc_edge(1, xl2, xr2, e2, src, dst, att2)

    lp, ent, val = _stage_e(acc2, den2, bias2.reshape(1, -1),
                            g2.reshape(1, -1), be2.reshape(1, -1), action,
                            log_std.reshape(1, -1), A1, a1.reshape(1, -1),
                            A2.reshape(1, -1), a2.reshape(1, 1), C1,
                            c1.reshape(1, -1), C2.reshape(1, -1),
                            c2.reshape(1, 1))

    return (action, lp.reshape(B), ent.reshape(B), val)
